# baseline (speedup 1.0000x reference)
import functools

import jax
import jax.numpy as jnp
import numpy as np
from jax import lax
from jax.experimental import pallas as pl
from jax.experimental.pallas import tpu as pltpu
from jax.experimental.pallas import tpu_sc as plsc

B = 16384
L = 50
EMB = 64
QW = 32
CW = 2 * L
NLANE = 16
NC, NS = 2, 16
NW = NC * NS
BPW = B // NW
NB = 8

VOCAB = 1000000
H1 = 90
H2 = 90
NOUT = 3
BB = 1024

TM = 4096
TNB = (VOCAB + 2 * TM - 1) // (2 * TM)
TLASTB = (VOCAB - 1) // TM
NPAIR = TNB * TM

_PERM = np.concatenate([np.arange(0, 32, 2), np.arange(1, 32, 2),
                        np.arange(32, 64, 2), np.arange(33, 64, 2)])


def _pool_side(r_hbm, out_hbm, tab_hbm, idx_v, rows_v, obuf_v, sems, wid):
    base = wid * BPW
    pltpu.sync_copy(r_hbm.at[pl.ds(base, BPW)], idx_v)

    for b in range(NB):
        pltpu.async_copy(tab_hbm.at[idx_v.at[b]], rows_v.at[b], sems.at[b])

    def process(k):
        b = jnp.bitwise_and(k, NB - 1)
        pltpu.make_async_copy(tab_hbm.at[idx_v.at[k]], rows_v.at[b],
                              sems.at[b]).wait()
        acc = [None] * 4
        for r in range(L):
            for t in range(2):
                x = rows_v[b, 2 * r + t]
                lo, hi = plsc.unpack(x, format=plsc.PackFormat.INTERLEAVED)
                if r == 0:
                    acc[2 * t] = lo
                    acc[2 * t + 1] = hi
                else:
                    acc[2 * t] = acc[2 * t] + lo
                    acc[2 * t + 1] = acc[2 * t + 1] + hi
        for c in range(4):
            obuf_v[k, pl.ds(c * NLANE, NLANE)] = acc[c] * (1.0 / L)

    @pl.loop(0, BPW - NB)
    def _(k):
        process(k)
        bb = jnp.bitwise_and(k, NB - 1)
        pltpu.async_copy(tab_hbm.at[idx_v.at[k + NB]], rows_v.at[bb],
                         sems.at[bb])

    @pl.loop(BPW - NB, BPW)
    def _(k):
        process(k)

    pltpu.sync_copy(obuf_v, out_hbm.at[pl.ds(base, BPW)])


@functools.cache
def _make_pool_sc():
    @functools.partial(
        pl.kernel,
        out_type=(
            jax.ShapeDtypeStruct((B, EMB), jnp.float32),
            jax.ShapeDtypeStruct((B, EMB), jnp.float32),
        ),
        mesh=plsc.VectorSubcoreMesh(core_axis_name="c", subcore_axis_name="s",
                                    num_cores=NC, num_subcores=NS),
        scratch_types=(
            pltpu.VMEM((BPW, CW), jnp.int32),
            pltpu.VMEM((NB, CW, QW), jnp.bfloat16),
            pltpu.VMEM((BPW, EMB), jnp.float32),
            pltpu.SemaphoreType.DMA((NB,)),
        ),
        compiler_params=pltpu.CompilerParams(use_tc_tiling_on_sc=False,
                                             needs_layout_passes=False),
    )
    def _pool_sc(rh_hbm, rp_hbm, tab_hbm, outh_hbm, outp_hbm,
                 idx_v, rows_v, obuf_v, sems):
        wid = lax.axis_index("s") * NC + lax.axis_index("c")
        _pool_side(rh_hbm, outh_hbm, tab_hbm, idx_v, rows_v, obuf_v, sems, wid)
        _pool_side(rp_hbm, outp_hbm, tab_hbm, idx_v, rows_v, obuf_v, sems, wid)

    return _pool_sc


def _tr_body(a_ref, b_ref, o_ref):
    o_ref[...] = jnp.concatenate(
        [jnp.transpose(a_ref[...]), jnp.transpose(b_ref[...])],
        axis=1).astype(jnp.bfloat16)


def _relayout_tc(tt):
    return pl.pallas_call(
        _tr_body,
        grid=(TNB,),
        in_specs=[
            pl.BlockSpec((EMB, TM), lambda i: (0, jnp.minimum(2 * i, TLASTB))),
            pl.BlockSpec((EMB, TM),
                         lambda i: (0, jnp.minimum(2 * i + 1, TLASTB))),
        ],
        out_specs=pl.BlockSpec((TM, 2 * EMB), lambda i: (i, 0)),
        out_shape=jax.ShapeDtypeStruct((NPAIR, 2 * EMB), jnp.bfloat16),
    )(tt, tt)


def _mlp_body(h_ref, p_ref, w1a_ref, w1b_ref, b1_ref, w2_ref, b2_ref,
              w3_ref, b3_ref, y_ref):
    x = (jnp.dot(h_ref[...], w1a_ref[...], preferred_element_type=jnp.float32)
         + jnp.dot(p_ref[...], w1b_ref[...], preferred_element_type=jnp.float32)
         + b1_ref[...])
    x = jnp.maximum(x, 0.0)
    x = jnp.dot(x, w2_ref[...], preferred_element_type=jnp.float32) + b2_ref[...]
    x = jnp.maximum(x, 0.0)
    y_ref[...] = jnp.dot(x, w3_ref[...], preferred_element_type=jnp.float32) + b3_ref[...]


def _mlp_tc(ph, pp, w1a, w1b, b1, w2, b2, w3, b3):
    grid = (B // BB,)
    return pl.pallas_call(
        _mlp_body,
        grid=grid,
        in_specs=[
            pl.BlockSpec((BB, EMB), lambda i: (i, 0)),
            pl.BlockSpec((BB, EMB), lambda i: (i, 0)),
            pl.BlockSpec((EMB, H1), lambda i: (0, 0)),
            pl.BlockSpec((EMB, H1), lambda i: (0, 0)),
            pl.BlockSpec((1, H1), lambda i: (0, 0)),
            pl.BlockSpec((H1, H2), lambda i: (0, 0)),
            pl.BlockSpec((1, H2), lambda i: (0, 0)),
            pl.BlockSpec((H2, NOUT), lambda i: (0, 0)),
            pl.BlockSpec((1, NOUT), lambda i: (0, 0)),
        ],
        out_specs=pl.BlockSpec((BB, NOUT), lambda i: (i, 0)),
        out_shape=jax.ShapeDtypeStruct((B, NOUT), jnp.float32),
    )(ph, pp, w1a, w1b, b1, w2, b2, w3, b3)


def kernel(data_hypo, length_hypo, data_prem, length_prem, table,
           W1, b1, W2, b2, W3, b3):
    def qidx(d):
        d = d.astype(jnp.int32)
        p = (d >> 13) * TM + (d & (TM - 1))
        q0 = (p << 2) + (((d >> 12) & 1) << 1)
        return jnp.stack([q0, q0 + 1], axis=-1).reshape(B, CW)

    rh = qidx(data_hypo)
    rp = qidx(data_prem)
    tab_q = jnp.reshape(_relayout_tc(jnp.transpose(table)), (4 * NPAIR, QW))
    ph, pp = _make_pool_sc()(rh, rp, tab_q)
    perm = jnp.asarray(_PERM)
    w1a = jnp.take(W1[:EMB], perm, axis=0)
    w1b = jnp.take(W1[EMB:], perm, axis=0)
    return _mlp_tc(ph, pp, w1a, w1b, b1[None, :], W2, b2[None, :],
                   W3, b3[None, :])


# speedup vs baseline: 1.5174x; 1.5174x over previous
import functools

import jax
import jax.numpy as jnp
from jax import lax
from jax.experimental import pallas as pl
from jax.experimental.pallas import tpu as pltpu
from jax.experimental.pallas import tpu_sc as plsc

B = 16384
L = 50
EMB = 64
QW = 32
CW = 2 * L
NLANE = 16
NC, NS = 2, 16
NW = NC * NS
BPW = B // NW
NB = 8

VOCAB = 1000000
H1 = 90
H2 = 90
NOUT = 3
BB = 1024

TM = 4096
TNB = (VOCAB + 2 * TM - 1) // (2 * TM)
TLASTB = (VOCAB - 1) // TM
NPAIR = TNB * TM


def _pool_side(r_hbm, out_hbm, tab_hbm, idx_v, rows_v, obuf_v, sems, wid):
    base = wid * BPW
    pltpu.sync_copy(r_hbm.at[pl.ds(base, BPW)], idx_v)

    for b in range(NB):
        pltpu.async_copy(tab_hbm.at[idx_v.at[b]], rows_v.at[b], sems.at[b])

    def process(k):
        b = jnp.bitwise_and(k, NB - 1)
        pltpu.make_async_copy(tab_hbm.at[idx_v.at[k]], rows_v.at[b],
                              sems.at[b]).wait()
        acc = [rows_v[b, c // 2, pl.ds((c % 2) * NLANE, NLANE)]
               for c in range(EMB // NLANE)]
        for r in range(1, L):
            for c in range(EMB // NLANE):
                acc[c] = acc[c] + rows_v[b, 2 * r + c // 2,
                                         pl.ds((c % 2) * NLANE, NLANE)]
        for c in range(EMB // NLANE):
            obuf_v[k, pl.ds(c * NLANE, NLANE)] = acc[c] * (1.0 / L)

    @pl.loop(0, BPW - NB)
    def _(k):
        process(k)
        bb = jnp.bitwise_and(k, NB - 1)
        pltpu.async_copy(tab_hbm.at[idx_v.at[k + NB]], rows_v.at[bb],
                         sems.at[bb])

    @pl.loop(BPW - NB, BPW)
    def _(k):
        process(k)

    pltpu.sync_copy(obuf_v, out_hbm.at[pl.ds(base, BPW)])


@functools.cache
def _make_pool_sc():
    @functools.partial(
        pl.kernel,
        out_type=(
            jax.ShapeDtypeStruct((B, EMB), jnp.float32),
            jax.ShapeDtypeStruct((B, EMB), jnp.float32),
        ),
        mesh=plsc.VectorSubcoreMesh(core_axis_name="c", subcore_axis_name="s",
                                    num_cores=NC, num_subcores=NS),
        scratch_types=(
            pltpu.VMEM((BPW, CW), jnp.int32),
            pltpu.VMEM((NB, CW, QW), jnp.float32),
            pltpu.VMEM((BPW, EMB), jnp.float32),
            pltpu.SemaphoreType.DMA((NB,)),
        ),
        compiler_params=pltpu.CompilerParams(use_tc_tiling_on_sc=False),
    )
    def _pool_sc(rh_hbm, rp_hbm, tab_hbm, outh_hbm, outp_hbm,
                 idx_v, rows_v, obuf_v, sems):
        wid = lax.axis_index("s") * NC + lax.axis_index("c")
        _pool_side(rh_hbm, outh_hbm, tab_hbm, idx_v, rows_v, obuf_v, sems, wid)
        _pool_side(rp_hbm, outp_hbm, tab_hbm, idx_v, rows_v, obuf_v, sems, wid)

    return _pool_sc


def _tr_body(a_ref, b_ref, o_ref):
    a = jnp.transpose(a_ref[...])
    b = jnp.transpose(b_ref[...])
    o_ref[...] = jnp.concatenate([a, b], axis=1)


def _relayout_tc(tt):
    return pl.pallas_call(
        _tr_body,
        grid=(TNB,),
        in_specs=[
            pl.BlockSpec((EMB, TM), lambda i: (0, jnp.minimum(2 * i, TLASTB))),
            pl.BlockSpec((EMB, TM),
                         lambda i: (0, jnp.minimum(2 * i + 1, TLASTB))),
        ],
        out_specs=pl.BlockSpec((TM, 2 * EMB), lambda i: (i, 0)),
        out_shape=jax.ShapeDtypeStruct((NPAIR, 2 * EMB), jnp.float32),
    )(tt, tt)


def _mlp_body(h_ref, p_ref, w1a_ref, w1b_ref, b1_ref, w2_ref, b2_ref,
              w3_ref, b3_ref, y_ref):
    x = (jnp.dot(h_ref[...], w1a_ref[...], preferred_element_type=jnp.float32)
         + jnp.dot(p_ref[...], w1b_ref[...], preferred_element_type=jnp.float32)
         + b1_ref[...])
    x = jnp.maximum(x, 0.0)
    x = jnp.dot(x, w2_ref[...], preferred_element_type=jnp.float32) + b2_ref[...]
    x = jnp.maximum(x, 0.0)
    y_ref[...] = jnp.dot(x, w3_ref[...], preferred_element_type=jnp.float32) + b3_ref[...]


def _mlp_tc(ph, pp, w1a, w1b, b1, w2, b2, w3, b3):
    grid = (B // BB,)
    return pl.pallas_call(
        _mlp_body,
        grid=grid,
        in_specs=[
            pl.BlockSpec((BB, EMB), lambda i: (i, 0)),
            pl.BlockSpec((BB, EMB), lambda i: (i, 0)),
            pl.BlockSpec((EMB, H1), lambda i: (0, 0)),
            pl.BlockSpec((EMB, H1), lambda i: (0, 0)),
            pl.BlockSpec((1, H1), lambda i: (0, 0)),
            pl.BlockSpec((H1, H2), lambda i: (0, 0)),
            pl.BlockSpec((1, H2), lambda i: (0, 0)),
            pl.BlockSpec((H2, NOUT), lambda i: (0, 0)),
            pl.BlockSpec((1, NOUT), lambda i: (0, 0)),
        ],
        out_specs=pl.BlockSpec((BB, NOUT), lambda i: (i, 0)),
        out_shape=jax.ShapeDtypeStruct((B, NOUT), jnp.float32),
    )(ph, pp, w1a, w1b, b1, w2, b2, w3, b3)


def kernel(data_hypo, length_hypo, data_prem, length_prem, table,
           W1, b1, W2, b2, W3, b3):
    def qidx(d):
        d = d.astype(jnp.int32)
        p = (d >> 13) * TM + (d & (TM - 1))
        q0 = (p << 2) + (((d >> 12) & 1) << 1)
        return jnp.stack([q0, q0 + 1], axis=-1).reshape(B, CW)

    rh = qidx(data_hypo)
    rp = qidx(data_prem)
    tab_q = jnp.reshape(_relayout_tc(jnp.transpose(table)), (4 * NPAIR, QW))
    ph, pp = _make_pool_sc()(rh, rp, tab_q)
    return _mlp_tc(ph, pp, W1[:EMB], W1[EMB:], b1[None, :], W2, b2[None, :],
                   W3, b3[None, :])


# speedup vs baseline: 1.6085x; 1.0600x over previous
import functools

import jax
import jax.numpy as jnp
from jax import lax
from jax.experimental import pallas as pl
from jax.experimental.pallas import tpu as pltpu
from jax.experimental.pallas import tpu_sc as plsc

B = 16384
L = 50
EMB = 64
QW = 32
CW = 2 * L
NLANE = 16
NC, NS = 2, 16
NW = NC * NS
BPW = B // NW
NB = 8

VOCAB = 1000000
H1 = 90
H2 = 90
NOUT = 3
BB = 1024

TM = 8192
TNB = (VOCAB + 2 * TM - 1) // (2 * TM)
TLASTB = (VOCAB - 1) // TM
NPAIR = TNB * TM


def _pool_side(r_hbm, out_hbm, tab_hbm, idx_v, rows_v, obuf_v, sems, wid):
    base = wid * BPW
    pltpu.sync_copy(r_hbm.at[pl.ds(base, BPW)], idx_v)

    for b in range(NB):
        pltpu.async_copy(tab_hbm.at[idx_v.at[b]], rows_v.at[b], sems.at[b])

    def process(k):
        b = jnp.bitwise_and(k, NB - 1)
        pltpu.make_async_copy(tab_hbm.at[idx_v.at[k]], rows_v.at[b],
                              sems.at[b]).wait()
        acc = [rows_v[b, c // 2, pl.ds((c % 2) * NLANE, NLANE)]
               for c in range(EMB // NLANE)]
        for r in range(1, L):
            for c in range(EMB // NLANE):
                acc[c] = acc[c] + rows_v[b, 2 * r + c // 2,
                                         pl.ds((c % 2) * NLANE, NLANE)]
        for c in range(EMB // NLANE):
            obuf_v[k, pl.ds(c * NLANE, NLANE)] = acc[c] * (1.0 / L)

    @pl.loop(0, BPW - NB)
    def _(k):
        process(k)
        bb = jnp.bitwise_and(k, NB - 1)
        pltpu.async_copy(tab_hbm.at[idx_v.at[k + NB]], rows_v.at[bb],
                         sems.at[bb])

    @pl.loop(BPW - NB, BPW)
    def _(k):
        process(k)

    pltpu.sync_copy(obuf_v, out_hbm.at[pl.ds(base, BPW)])


@functools.cache
def _make_pool_sc():
    @functools.partial(
        pl.kernel,
        out_type=(
            jax.ShapeDtypeStruct((B, EMB), jnp.float32),
            jax.ShapeDtypeStruct((B, EMB), jnp.float32),
        ),
        mesh=plsc.VectorSubcoreMesh(core_axis_name="c", subcore_axis_name="s",
                                    num_cores=NC, num_subcores=NS),
        scratch_types=(
            pltpu.VMEM((BPW, CW), jnp.int32),
            pltpu.VMEM((NB, CW, QW), jnp.float32),
            pltpu.VMEM((BPW, EMB), jnp.float32),
            pltpu.SemaphoreType.DMA((NB,)),
        ),
        compiler_params=pltpu.CompilerParams(use_tc_tiling_on_sc=False),
    )
    def _pool_sc(rh_hbm, rp_hbm, tab_hbm, outh_hbm, outp_hbm,
                 idx_v, rows_v, obuf_v, sems):
        wid = lax.axis_index("s") * NC + lax.axis_index("c")
        _pool_side(rh_hbm, outh_hbm, tab_hbm, idx_v, rows_v, obuf_v, sems, wid)
        _pool_side(rp_hbm, outp_hbm, tab_hbm, idx_v, rows_v, obuf_v, sems, wid)

    return _pool_sc


def _tr_body(a_ref, b_ref, o_ref):
    a = jnp.transpose(a_ref[...])
    b = jnp.transpose(b_ref[...])
    o_ref[...] = jnp.concatenate([a, b], axis=1)


def _relayout_tc(tt):
    return pl.pallas_call(
        _tr_body,
        grid=(TNB,),
        in_specs=[
            pl.BlockSpec((EMB, TM), lambda i: (0, jnp.minimum(2 * i, TLASTB))),
            pl.BlockSpec((EMB, TM),
                         lambda i: (0, jnp.minimum(2 * i + 1, TLASTB))),
        ],
        out_specs=pl.BlockSpec((TM, 2 * EMB), lambda i: (i, 0)),
        out_shape=jax.ShapeDtypeStruct((NPAIR, 2 * EMB), jnp.float32),
    )(tt, tt)


def _mlp_body(h_ref, p_ref, w1a_ref, w1b_ref, b1_ref, w2_ref, b2_ref,
              w3_ref, b3_ref, y_ref):
    x = (jnp.dot(h_ref[...], w1a_ref[...], preferred_element_type=jnp.float32)
         + jnp.dot(p_ref[...], w1b_ref[...], preferred_element_type=jnp.float32)
         + b1_ref[...])
    x = jnp.maximum(x, 0.0)
    x = jnp.dot(x, w2_ref[...], preferred_element_type=jnp.float32) + b2_ref[...]
    x = jnp.maximum(x, 0.0)
    y_ref[...] = jnp.dot(x, w3_ref[...], preferred_element_type=jnp.float32) + b3_ref[...]


def _mlp_tc(ph, pp, w1a, w1b, b1, w2, b2, w3, b3):
    grid = (B // BB,)
    return pl.pallas_call(
        _mlp_body,
        grid=grid,
        in_specs=[
            pl.BlockSpec((BB, EMB), lambda i: (i, 0)),
            pl.BlockSpec((BB, EMB), lambda i: (i, 0)),
            pl.BlockSpec((EMB, H1), lambda i: (0, 0)),
            pl.BlockSpec((EMB, H1), lambda i: (0, 0)),
            pl.BlockSpec((1, H1), lambda i: (0, 0)),
            pl.BlockSpec((H1, H2), lambda i: (0, 0)),
            pl.BlockSpec((1, H2), lambda i: (0, 0)),
            pl.BlockSpec((H2, NOUT), lambda i: (0, 0)),
            pl.BlockSpec((1, NOUT), lambda i: (0, 0)),
        ],
        out_specs=pl.BlockSpec((BB, NOUT), lambda i: (i, 0)),
        out_shape=jax.ShapeDtypeStruct((B, NOUT), jnp.float32),
    )(ph, pp, w1a, w1b, b1, w2, b2, w3, b3)


def kernel(data_hypo, length_hypo, data_prem, length_prem, table,
           W1, b1, W2, b2, W3, b3):
    def qidx(d):
        d = d.astype(jnp.int32)
        p = (d >> 14) * TM + (d & (TM - 1))
        q0 = (p << 2) + (((d >> 13) & 1) << 1)
        return jnp.stack([q0, q0 + 1], axis=-1).reshape(B, CW)

    rh = qidx(data_hypo)
    rp = qidx(data_prem)
    tab_q = jnp.reshape(_relayout_tc(jnp.transpose(table)), (4 * NPAIR, QW))
    ph, pp = _make_pool_sc()(rh, rp, tab_q)
    return _mlp_tc(ph, pp, W1[:EMB], W1[EMB:], b1[None, :], W2, b2[None, :],
                   W3, b3[None, :])


# speedup vs baseline: 1.6640x; 1.0345x over previous
import functools

import jax
import jax.numpy as jnp
from jax import lax
from jax.experimental import pallas as pl
from jax.experimental.pallas import tpu as pltpu
from jax.experimental.pallas import tpu_sc as plsc

B = 16384
L = 50
EMB = 64
QW = 32
CW = 2 * L
NLANE = 16
NC, NS = 2, 16
NW = NC * NS
BPW = B // NW
NB = 8

VOCAB = 1000000
H1 = 90
H2 = 90
NOUT = 3
BB = 1024

TM = 16384
TNB = (VOCAB + 2 * TM - 1) // (2 * TM)
TLASTB = (VOCAB - 1) // TM
NPAIR = TNB * TM


def _pool_side(r_hbm, out_hbm, tab_hbm, idx_v, rows_v, obuf_v, sems, wid):
    base = wid * BPW
    pltpu.sync_copy(r_hbm.at[pl.ds(base, BPW)], idx_v)

    for b in range(NB):
        pltpu.async_copy(tab_hbm.at[idx_v.at[b]], rows_v.at[b], sems.at[b])

    def process(k):
        b = jnp.bitwise_and(k, NB - 1)
        pltpu.make_async_copy(tab_hbm.at[idx_v.at[k]], rows_v.at[b],
                              sems.at[b]).wait()
        acc = [rows_v[b, c // 2, pl.ds((c % 2) * NLANE, NLANE)]
               for c in range(EMB // NLANE)]
        for r in range(1, L):
            for c in range(EMB // NLANE):
                acc[c] = acc[c] + rows_v[b, 2 * r + c // 2,
                                         pl.ds((c % 2) * NLANE, NLANE)]
        for c in range(EMB // NLANE):
            obuf_v[k, pl.ds(c * NLANE, NLANE)] = acc[c] * (1.0 / L)

    @pl.loop(0, BPW - NB)
    def _(k):
        process(k)
        bb = jnp.bitwise_and(k, NB - 1)
        pltpu.async_copy(tab_hbm.at[idx_v.at[k + NB]], rows_v.at[bb],
                         sems.at[bb])

    @pl.loop(BPW - NB, BPW)
    def _(k):
        process(k)

    pltpu.sync_copy(obuf_v, out_hbm.at[pl.ds(base, BPW)])


@functools.cache
def _make_pool_sc():
    @functools.partial(
        pl.kernel,
        out_type=(
            jax.ShapeDtypeStruct((B, EMB), jnp.float32),
            jax.ShapeDtypeStruct((B, EMB), jnp.float32),
        ),
        mesh=plsc.VectorSubcoreMesh(core_axis_name="c", subcore_axis_name="s",
                                    num_cores=NC, num_subcores=NS),
        scratch_types=(
            pltpu.VMEM((BPW, CW), jnp.int32),
            pltpu.VMEM((NB, CW, QW), jnp.float32),
            pltpu.VMEM((BPW, EMB), jnp.float32),
            pltpu.SemaphoreType.DMA((NB,)),
        ),
        compiler_params=pltpu.CompilerParams(use_tc_tiling_on_sc=False),
    )
    def _pool_sc(rh_hbm, rp_hbm, tab_hbm, outh_hbm, outp_hbm,
                 idx_v, rows_v, obuf_v, sems):
        wid = lax.axis_index("s") * NC + lax.axis_index("c")
        _pool_side(rh_hbm, outh_hbm, tab_hbm, idx_v, rows_v, obuf_v, sems, wid)
        _pool_side(rp_hbm, outp_hbm, tab_hbm, idx_v, rows_v, obuf_v, sems, wid)

    return _pool_sc


def _tr_body(a_ref, b_ref, o_ref):
    a = jnp.transpose(a_ref[...])
    b = jnp.transpose(b_ref[...])
    o_ref[...] = jnp.concatenate([a, b], axis=1)


def _relayout_tc(tt):
    return pl.pallas_call(
        _tr_body,
        grid=(TNB,),
        in_specs=[
            pl.BlockSpec((EMB, TM), lambda i: (0, jnp.minimum(2 * i, TLASTB))),
            pl.BlockSpec((EMB, TM),
                         lambda i: (0, jnp.minimum(2 * i + 1, TLASTB))),
        ],
        out_specs=pl.BlockSpec((TM, 2 * EMB), lambda i: (i, 0)),
        out_shape=jax.ShapeDtypeStruct((NPAIR, 2 * EMB), jnp.float32),
    )(tt, tt)


def _mlp_body(h_ref, p_ref, w1a_ref, w1b_ref, b1_ref, w2_ref, b2_ref,
              w3_ref, b3_ref, y_ref):
    x = (jnp.dot(h_ref[...], w1a_ref[...], preferred_element_type=jnp.float32)
         + jnp.dot(p_ref[...], w1b_ref[...], preferred_element_type=jnp.float32)
         + b1_ref[...])
    x = jnp.maximum(x, 0.0)
    x = jnp.dot(x, w2_ref[...], preferred_element_type=jnp.float32) + b2_ref[...]
    x = jnp.maximum(x, 0.0)
    y_ref[...] = jnp.dot(x, w3_ref[...], preferred_element_type=jnp.float32) + b3_ref[...]


def _mlp_tc(ph, pp, w1a, w1b, b1, w2, b2, w3, b3):
    grid = (B // BB,)
    return pl.pallas_call(
        _mlp_body,
        grid=grid,
        in_specs=[
            pl.BlockSpec((BB, EMB), lambda i: (i, 0)),
            pl.BlockSpec((BB, EMB), lambda i: (i, 0)),
            pl.BlockSpec((EMB, H1), lambda i: (0, 0)),
            pl.BlockSpec((EMB, H1), lambda i: (0, 0)),
            pl.BlockSpec((1, H1), lambda i: (0, 0)),
            pl.BlockSpec((H1, H2), lambda i: (0, 0)),
            pl.BlockSpec((1, H2), lambda i: (0, 0)),
            pl.BlockSpec((H2, NOUT), lambda i: (0, 0)),
            pl.BlockSpec((1, NOUT), lambda i: (0, 0)),
        ],
        out_specs=pl.BlockSpec((BB, NOUT), lambda i: (i, 0)),
        out_shape=jax.ShapeDtypeStruct((B, NOUT), jnp.float32),
    )(ph, pp, w1a, w1b, b1, w2, b2, w3, b3)


def kernel(data_hypo, length_hypo, data_prem, length_prem, table,
           W1, b1, W2, b2, W3, b3):
    def qidx(d):
        d = d.astype(jnp.int32)
        p = (d >> 15) * TM + (d & (TM - 1))
        q0 = (p << 2) + (((d >> 14) & 1) << 1)
        return jnp.stack([q0, q0 + 1], axis=-1).reshape(B, CW)

    rh = qidx(data_hypo)
    rp = qidx(data_prem)
    tab_q = jnp.reshape(_relayout_tc(jnp.transpose(table)), (4 * NPAIR, QW))
    ph, pp = _make_pool_sc()(rh, rp, tab_q)
    return _mlp_tc(ph, pp, W1[:EMB], W1[EMB:], b1[None, :], W2, b2[None, :],
                   W3, b3[None, :])
